# Initial kernel scaffold; baseline (speedup 1.0000x reference)
import jax
import jax.numpy as jnp
from jax.experimental import pallas as pl


def kernel(x, edge_index, W, b):
    raise NotImplementedError("write your pallas kernel here")


# speedup vs baseline: 21.0887x; 21.0887x over previous
import functools

import jax
import jax.numpy as jnp
from jax import lax
from jax.experimental import pallas as pl
from jax.experimental.pallas import tpu as pltpu
from jax.experimental.pallas import tpu_sc as plsc

_N = 10000
_E = 320000
_D = 128

_NC = 2
_NS = 16
_NW = _NC * _NS
_EPW = _E // _NW
_CH = 40
_NCHUNK = _EPW // _CH
_ROUNDS = ((0, 128), (128, 122))
_RMAX = 128
_ROWS_PER_TILE = _N // _NS
_DEGW = 128


def _deg_body(dst_hbm, out_hbm, idx_v, ones_v, zbuf_v, deg_sh, sem_a, sem_b):
  c = lax.axis_index("c")
  s = lax.axis_index("s")
  wid = c * _NS + s

  def fill(i, _):
    ones_v[i, :] = jnp.ones((_DEGW,), jnp.float32)
    return 0

  lax.fori_loop(0, _CH, fill, 0, unroll=4)

  def zfill(i, _):
    zbuf_v[i, :] = jnp.zeros((_DEGW,), jnp.float32)
    return 0

  lax.fori_loop(0, 25, zfill, 0, unroll=4)

  def zcopy(q, _):
    pltpu.sync_copy(
        zbuf_v, deg_sh.at[pl.ds(s * _ROWS_PER_TILE + q * 25, 25)])
    return 0

  lax.fori_loop(0, _ROWS_PER_TILE // 25, zcopy, 0)
  plsc.subcore_barrier()

  def start(j, sem):
    pltpu.async_copy(ones_v, deg_sh.at[idx_v.at[j]], sem, add=True)

  def wait(j, sem):
    pltpu.make_async_copy(ones_v, deg_sh.at[idx_v.at[j]], sem).wait()

  for off, nch in _ROUNDS:
    pltpu.sync_copy(dst_hbm.at[wid, pl.ds(off, nch)],
                    idx_v.at[pl.ds(0, nch)])
    npair = nch // 2
    start(0, sem_a)
    start(1, sem_b)

    def body(k, _):
      j = 2 * k
      wait(j, sem_a)

      @pl.when(k < npair - 1)
      def _():
        start(j + 2, sem_a)

      wait(j + 1, sem_b)

      @pl.when(k < npair - 1)
      def _():
        start(j + 3, sem_b)

      return 0

    lax.fori_loop(0, npair, body, 0)

  plsc.subcore_barrier()
  pltpu.sync_copy(
      deg_sh.at[pl.ds(s * _ROWS_PER_TILE, _ROWS_PER_TILE)],
      out_hbm.at[c, s],
  )


@functools.lru_cache(maxsize=None)
def _deg_kernel():
  return pl.kernel(
      _deg_body,
      out_type=jax.ShapeDtypeStruct(
          (_NC, _NS, _ROWS_PER_TILE, _DEGW), jnp.float32),
      mesh=plsc.VectorSubcoreMesh(
          core_axis_name="c", subcore_axis_name="s",
          num_cores=_NC, num_subcores=_NS),
      scratch_types=[
          pltpu.VMEM((_RMAX, _CH), jnp.int32),
          pltpu.VMEM((_CH, _DEGW), jnp.float32),
          pltpu.VMEM((25, _DEGW), jnp.float32),
          pltpu.VMEM_SHARED((_N, _DEGW), jnp.float32),
          pltpu.SemaphoreType.DMA,
          pltpu.SemaphoreType.DMA,
      ],
  )


def _edge_body(sup_hbm, src_hbm, dst_hbm, zeros_hbm, out_hbm, sidx_v, didx_v,
               rows_a, rows_b, acc_sh, sem_ga, sem_gb, sem_sa, sem_sb):
  c = lax.axis_index("c")
  s = lax.axis_index("s")
  wid = c * _NS + s

  pltpu.sync_copy(zeros_hbm,
                  acc_sh.at[pl.ds(s * _ROWS_PER_TILE, _ROWS_PER_TILE)])
  plsc.subcore_barrier()

  def gstart(j, buf, sem):
    pltpu.async_copy(sup_hbm.at[sidx_v.at[j]], buf, sem)

  def gwait(j, buf, sem):
    pltpu.make_async_copy(sup_hbm.at[sidx_v.at[j]], buf, sem).wait()

  def sstart(j, buf, sem):
    pltpu.async_copy(buf, acc_sh.at[didx_v.at[j]], sem, add=True)

  def swait(j, buf, sem):
    pltpu.make_async_copy(buf, acc_sh.at[didx_v.at[j]], sem).wait()

  for off, nch in _ROUNDS:
    pltpu.sync_copy(src_hbm.at[wid, pl.ds(off, nch)],
                    sidx_v.at[pl.ds(0, nch)])
    pltpu.sync_copy(dst_hbm.at[wid, pl.ds(off, nch)],
                    didx_v.at[pl.ds(0, nch)])
    npair = nch // 2
    gstart(0, rows_a, sem_ga)
    gstart(1, rows_b, sem_gb)

    def body(k, _):
      j = 2 * k
      gwait(j, rows_a, sem_ga)
      sstart(j, rows_a, sem_sa)
      gwait(j + 1, rows_b, sem_gb)
      sstart(j + 1, rows_b, sem_sb)
      swait(j, rows_a, sem_sa)

      @pl.when(k < npair - 1)
      def _():
        gstart(j + 2, rows_a, sem_ga)

      swait(j + 1, rows_b, sem_sb)

      @pl.when(k < npair - 1)
      def _():
        gstart(j + 3, rows_b, sem_gb)

      return 0

    lax.fori_loop(0, npair, body, 0)

  plsc.subcore_barrier()
  pltpu.sync_copy(
      acc_sh.at[pl.ds(s * _ROWS_PER_TILE, _ROWS_PER_TILE)],
      out_hbm.at[c, s],
  )


@functools.lru_cache(maxsize=None)
def _edge_kernel():
  return pl.kernel(
      _edge_body,
      out_type=jax.ShapeDtypeStruct(
          (_NC, _NS, _ROWS_PER_TILE, _D), jnp.float32),
      mesh=plsc.VectorSubcoreMesh(
          core_axis_name="c", subcore_axis_name="s",
          num_cores=_NC, num_subcores=_NS),
      scratch_types=[
          pltpu.VMEM((_RMAX, _CH), jnp.int32),
          pltpu.VMEM((_RMAX, _CH), jnp.int32),
          pltpu.VMEM((_CH, _D), jnp.float32),
          pltpu.VMEM((_CH, _D), jnp.float32),
          pltpu.VMEM_SHARED((_N, _D), jnp.float32),
          pltpu.SemaphoreType.DMA,
          pltpu.SemaphoreType.DMA,
          pltpu.SemaphoreType.DMA,
          pltpu.SemaphoreType.DMA,
      ],
  )


def _dis_from_parts(degp):
  deg = degp[0, :, 0] + degp[1, :, 0]
  return jnp.where(deg > 0, lax.rsqrt(deg), 0.0)


def _support_body(x_ref, w_ref, degp_ref, o_ref):
  sup = jnp.dot(x_ref[...], w_ref[...], preferred_element_type=jnp.float32)
  dis = _dis_from_parts(degp_ref[...])
  o_ref[...] = sup * dis[:, None]


def _support_kernel(x, w, degp):
  return pl.pallas_call(
      _support_body,
      out_shape=jax.ShapeDtypeStruct((_N, _D), jnp.float32),
  )(x, w, degp)


def _finish_body(accp_ref, degp_ref, b_ref, o_ref):
  acc = accp_ref[0] + accp_ref[1]
  dis = _dis_from_parts(degp_ref[...])
  o_ref[...] = jnp.maximum(acc * dis[:, None] + b_ref[...], 0.0)


def _finish_kernel(accp, degp, b2d):
  return pl.pallas_call(
      _finish_body,
      out_shape=jax.ShapeDtypeStruct((_N, _D), jnp.float32),
  )(accp, degp, b2d)


@jax.jit
def kernel(x, edge_index, W, b):
  src = edge_index[0].reshape(_NW, _NCHUNK, _CH)
  dst = edge_index[1].reshape(_NW, _NCHUNK, _CH)
  zeros128 = jnp.zeros((_ROWS_PER_TILE, _D), jnp.float32)
  degp = _deg_kernel()(dst).reshape(_NC, _N, _DEGW)
  sup2 = _support_kernel(x, W, degp)
  accp = _edge_kernel()(sup2, src, dst, zeros128).reshape(_NC, _N, _D)
  return _finish_kernel(accp, degp, b.reshape(1, _D))


# speedup vs baseline: 25.7711x; 1.2220x over previous
import functools

import jax
import jax.numpy as jnp
from jax import lax
from jax.experimental import pallas as pl
from jax.experimental.pallas import tpu as pltpu
from jax.experimental.pallas import tpu_sc as plsc

_N = 10000
_E = 320000
_D = 128

_NC = 2
_NS = 16
_NW = _NC * _NS
_EPW = _E // _NW
_CH = 40
_NCHUNK = _EPW // _CH
_ROUNDS = ((0, 128), (128, 122))
_RMAX = 128
_EROUNDS = ((0, 64), (64, 64), (128, 64), (192, 58))
_ERMAX = 64
_NBUF = 4
_ROWS_PER_TILE = _N // _NS
_DEGW = 128


def _deg_body(dst_hbm, out_hbm, idx_v, ones_v, zbuf_v, deg_sh, sem_a, sem_b):
  c = lax.axis_index("c")
  s = lax.axis_index("s")
  wid = c * _NS + s

  def fill(i, _):
    ones_v[i, :] = jnp.ones((_DEGW,), jnp.float32)
    return 0

  lax.fori_loop(0, _CH, fill, 0, unroll=4)

  def zfill(i, _):
    zbuf_v[i, :] = jnp.zeros((_DEGW,), jnp.float32)
    return 0

  lax.fori_loop(0, 25, zfill, 0, unroll=4)

  def zcopy(q, _):
    pltpu.sync_copy(
        zbuf_v, deg_sh.at[pl.ds(s * _ROWS_PER_TILE + q * 25, 25)])
    return 0

  lax.fori_loop(0, _ROWS_PER_TILE // 25, zcopy, 0)
  plsc.subcore_barrier()

  def start(j, sem):
    pltpu.async_copy(ones_v, deg_sh.at[idx_v.at[j]], sem, add=True)

  def wait(j, sem):
    pltpu.make_async_copy(ones_v, deg_sh.at[idx_v.at[j]], sem).wait()

  for off, nch in _ROUNDS:
    pltpu.sync_copy(dst_hbm.at[wid, pl.ds(off, nch)],
                    idx_v.at[pl.ds(0, nch)])
    npair = nch // 2
    start(0, sem_a)
    start(1, sem_b)

    def body(k, _):
      j = 2 * k
      wait(j, sem_a)

      @pl.when(k < npair - 1)
      def _():
        start(j + 2, sem_a)

      wait(j + 1, sem_b)

      @pl.when(k < npair - 1)
      def _():
        start(j + 3, sem_b)

      return 0

    lax.fori_loop(0, npair, body, 0)

  plsc.subcore_barrier()
  pltpu.sync_copy(
      deg_sh.at[pl.ds(s * _ROWS_PER_TILE, _ROWS_PER_TILE)],
      out_hbm.at[c, s],
  )


@functools.lru_cache(maxsize=None)
def _deg_kernel():
  return pl.kernel(
      _deg_body,
      out_type=jax.ShapeDtypeStruct(
          (_NC, _NS, _ROWS_PER_TILE, _DEGW), jnp.float32),
      mesh=plsc.VectorSubcoreMesh(
          core_axis_name="c", subcore_axis_name="s",
          num_cores=_NC, num_subcores=_NS),
      scratch_types=[
          pltpu.VMEM((_RMAX, _CH), jnp.int32),
          pltpu.VMEM((_CH, _DEGW), jnp.float32),
          pltpu.VMEM((25, _DEGW), jnp.float32),
          pltpu.VMEM_SHARED((_N, _DEGW), jnp.float32),
          pltpu.SemaphoreType.DMA,
          pltpu.SemaphoreType.DMA,
      ],
  )


def _edge_body(sup_hbm, src_hbm, dst_hbm, zeros_hbm, out_hbm, sidx_v, didx_v,
               rows0, rows1, rows2, rows3, acc_sh,
               gs0, gs1, gs2, gs3, ss0, ss1, ss2, ss3):
  c = lax.axis_index("c")
  s = lax.axis_index("s")
  wid = c * _NS + s
  rows = (rows0, rows1, rows2, rows3)
  gsem = (gs0, gs1, gs2, gs3)
  ssem = (ss0, ss1, ss2, ss3)

  pltpu.sync_copy(zeros_hbm,
                  acc_sh.at[pl.ds(s * _ROWS_PER_TILE, _ROWS_PER_TILE)])
  plsc.subcore_barrier()

  def gstart(j, b):
    pltpu.async_copy(sup_hbm.at[sidx_v.at[j]], rows[b], gsem[b])

  def gwait(j, b):
    pltpu.make_async_copy(sup_hbm.at[sidx_v.at[j]], rows[b], gsem[b]).wait()

  def sstart(j, b):
    pltpu.async_copy(rows[b], acc_sh.at[didx_v.at[j]], ssem[b], add=True)

  def swait(j, b):
    pltpu.make_async_copy(rows[b], acc_sh.at[didx_v.at[j]], ssem[b]).wait()

  for off, nch in _EROUNDS:
    pltpu.sync_copy(src_hbm.at[wid, pl.ds(off, nch)],
                    sidx_v.at[pl.ds(0, nch)])
    pltpu.sync_copy(dst_hbm.at[wid, pl.ds(off, nch)],
                    didx_v.at[pl.ds(0, nch)])
    nquad = nch // _NBUF
    tail = nch % _NBUF
    for b in range(_NBUF):
      gstart(b, b)

    def body(k, _):
      j = _NBUF * k
      for b in range(_NBUF):
        gwait(j + b, b)
        sstart(j + b, b)
      for b in range(_NBUF):
        swait(j + b, b)

        @pl.when(j + _NBUF + b < nch)
        def _():
          gstart(j + _NBUF + b, b)

      return 0

    lax.fori_loop(0, nquad, body, 0)
    for b in range(tail):
      j = nquad * _NBUF + b
      gwait(j, b)
      sstart(j, b)
      swait(j, b)

  plsc.subcore_barrier()
  pltpu.sync_copy(
      acc_sh.at[pl.ds(s * _ROWS_PER_TILE, _ROWS_PER_TILE)],
      out_hbm.at[c, s],
  )


@functools.lru_cache(maxsize=None)
def _edge_kernel():
  return pl.kernel(
      _edge_body,
      out_type=jax.ShapeDtypeStruct(
          (_NC, _NS, _ROWS_PER_TILE, _D), jnp.float32),
      mesh=plsc.VectorSubcoreMesh(
          core_axis_name="c", subcore_axis_name="s",
          num_cores=_NC, num_subcores=_NS),
      scratch_types=(
          [pltpu.VMEM((_ERMAX, _CH), jnp.int32)] * 2
          + [pltpu.VMEM((_CH, _D), jnp.float32)] * _NBUF
          + [pltpu.VMEM_SHARED((_N, _D), jnp.float32)]
          + [pltpu.SemaphoreType.DMA] * (2 * _NBUF)
      ),
  )


def _dis_from_parts(degp):
  deg = degp[0, :, 0] + degp[1, :, 0]
  return jnp.where(deg > 0, lax.rsqrt(deg), 0.0)


def _support_body(x_ref, w_ref, degp_ref, o_ref):
  sup = jnp.dot(x_ref[...], w_ref[...], preferred_element_type=jnp.float32)
  dis = _dis_from_parts(degp_ref[...])
  o_ref[...] = sup * dis[:, None]


def _support_kernel(x, w, degp):
  return pl.pallas_call(
      _support_body,
      out_shape=jax.ShapeDtypeStruct((_N, _D), jnp.float32),
  )(x, w, degp)


def _finish_body(accp_ref, degp_ref, b_ref, o_ref):
  acc = accp_ref[0] + accp_ref[1]
  dis = _dis_from_parts(degp_ref[...])
  o_ref[...] = jnp.maximum(acc * dis[:, None] + b_ref[...], 0.0)


def _finish_kernel(accp, degp, b2d):
  return pl.pallas_call(
      _finish_body,
      out_shape=jax.ShapeDtypeStruct((_N, _D), jnp.float32),
  )(accp, degp, b2d)


@jax.jit
def kernel(x, edge_index, W, b):
  src = edge_index[0].reshape(_NW, _NCHUNK, _CH)
  dst = edge_index[1].reshape(_NW, _NCHUNK, _CH)
  zeros128 = jnp.zeros((_ROWS_PER_TILE, _D), jnp.float32)
  degp = _deg_kernel()(dst).reshape(_NC, _N, _DEGW)
  sup2 = _support_kernel(x, W, degp)
  accp = _edge_kernel()(sup2, src, dst, zeros128).reshape(_NC, _N, _D)
  return _finish_kernel(accp, degp, b.reshape(1, _D))


# speedup vs baseline: 26.2761x; 1.0196x over previous
import functools

import jax
import jax.numpy as jnp
from jax import lax
from jax.experimental import pallas as pl
from jax.experimental.pallas import tpu as pltpu
from jax.experimental.pallas import tpu_sc as plsc

_N = 10000
_E = 320000
_D = 128

_NC = 2
_NS = 16
_NW = _NC * _NS
_EPW = _E // _NW
_CH = 40
_NCHUNK = _EPW // _CH
_ROUNDS = ((0, 128), (128, 122))
_RMAX = 128
_EROUNDS = ((0, 64), (64, 64), (128, 64), (192, 58))
_ERMAX = 64
_NBUF = 6
_ROWS_PER_TILE = _N // _NS
_DEGW = 128


def _deg_body(dst_hbm, out_hbm, idx_v, ones_v, zbuf_v, deg_sh,
              sem_a, sem_b, sem_c, sem_d):
  c = lax.axis_index("c")
  s = lax.axis_index("s")
  wid = c * _NS + s

  def fill(i, _):
    ones_v[i, :] = jnp.ones((_DEGW,), jnp.float32)
    return 0

  lax.fori_loop(0, _CH, fill, 0, unroll=4)

  def zfill(i, _):
    zbuf_v[i, :] = jnp.zeros((_DEGW,), jnp.float32)
    return 0

  lax.fori_loop(0, 25, zfill, 0, unroll=4)

  def zcopy(q, _):
    pltpu.sync_copy(
        zbuf_v, deg_sh.at[pl.ds(s * _ROWS_PER_TILE + q * 25, 25)])
    return 0

  lax.fori_loop(0, _ROWS_PER_TILE // 25, zcopy, 0)
  plsc.subcore_barrier()

  sems = (sem_a, sem_b, sem_c, sem_d)

  def start(j, b):
    pltpu.async_copy(ones_v, deg_sh.at[idx_v.at[j]], sems[b], add=True)

  def wait(j, b):
    pltpu.make_async_copy(ones_v, deg_sh.at[idx_v.at[j]], sems[b]).wait()

  for off, nch in _EROUNDS:
    pltpu.sync_copy(dst_hbm.at[wid, pl.ds(off, nch)],
                    idx_v.at[pl.ds(0, nch)])
    nquad = nch // 4
    tail = nch % 4
    for b in range(4):
      start(b, b)

    def body(k, _):
      j = 4 * k
      for b in range(4):
        wait(j + b, b)

        @pl.when(j + 4 + b < nch)
        def _():
          start(j + 4 + b, b)

      return 0

    lax.fori_loop(0, nquad, body, 0)
    for b in range(tail):
      wait(nquad * 4 + b, b)

  plsc.subcore_barrier()
  pltpu.sync_copy(
      deg_sh.at[pl.ds(s * _ROWS_PER_TILE, _ROWS_PER_TILE)],
      out_hbm.at[c, s],
  )


@functools.lru_cache(maxsize=None)
def _deg_kernel():
  return pl.kernel(
      _deg_body,
      out_type=jax.ShapeDtypeStruct(
          (_NC, _NS, _ROWS_PER_TILE, _DEGW), jnp.float32),
      mesh=plsc.VectorSubcoreMesh(
          core_axis_name="c", subcore_axis_name="s",
          num_cores=_NC, num_subcores=_NS),
      scratch_types=[
          pltpu.VMEM((_ERMAX, _CH), jnp.int32),
          pltpu.VMEM((_CH, _DEGW), jnp.float32),
          pltpu.VMEM((25, _DEGW), jnp.float32),
          pltpu.VMEM_SHARED((_N, _DEGW), jnp.float32),
          pltpu.SemaphoreType.DMA,
          pltpu.SemaphoreType.DMA,
          pltpu.SemaphoreType.DMA,
          pltpu.SemaphoreType.DMA,
      ],
  )


def _edge_body(sup_hbm, src_hbm, dst_hbm, zeros_hbm, out_hbm, sidx_v, didx_v,
               rows0, rows1, rows2, rows3, rows4, rows5, acc_sh,
               gs0, gs1, gs2, gs3, gs4, gs5, ss0, ss1, ss2, ss3, ss4, ss5):
  c = lax.axis_index("c")
  s = lax.axis_index("s")
  wid = c * _NS + s
  rows = (rows0, rows1, rows2, rows3, rows4, rows5)
  gsem = (gs0, gs1, gs2, gs3, gs4, gs5)
  ssem = (ss0, ss1, ss2, ss3, ss4, ss5)

  pltpu.sync_copy(zeros_hbm,
                  acc_sh.at[pl.ds(s * _ROWS_PER_TILE, _ROWS_PER_TILE)])
  plsc.subcore_barrier()

  def gstart(j, b):
    pltpu.async_copy(sup_hbm.at[sidx_v.at[j]], rows[b], gsem[b])

  def gwait(j, b):
    pltpu.make_async_copy(sup_hbm.at[sidx_v.at[j]], rows[b], gsem[b]).wait()

  def sstart(j, b):
    pltpu.async_copy(rows[b], acc_sh.at[didx_v.at[j]], ssem[b], add=True)

  def swait(j, b):
    pltpu.make_async_copy(rows[b], acc_sh.at[didx_v.at[j]], ssem[b]).wait()

  for off, nch in _EROUNDS:
    pltpu.sync_copy(src_hbm.at[wid, pl.ds(off, nch)],
                    sidx_v.at[pl.ds(0, nch)])
    pltpu.sync_copy(dst_hbm.at[wid, pl.ds(off, nch)],
                    didx_v.at[pl.ds(0, nch)])
    nquad = nch // _NBUF
    tail = nch % _NBUF
    for b in range(_NBUF):
      gstart(b, b)

    def body(k, _):
      j = _NBUF * k
      for b in range(_NBUF):
        gwait(j + b, b)
        sstart(j + b, b)
      for b in range(_NBUF):
        swait(j + b, b)

        @pl.when(j + _NBUF + b < nch)
        def _():
          gstart(j + _NBUF + b, b)

      return 0

    lax.fori_loop(0, nquad, body, 0)
    for b in range(tail):
      j = nquad * _NBUF + b
      gwait(j, b)
      sstart(j, b)
      swait(j, b)

  plsc.subcore_barrier()
  pltpu.sync_copy(
      acc_sh.at[pl.ds(s * _ROWS_PER_TILE, _ROWS_PER_TILE)],
      out_hbm.at[c, s],
  )


@functools.lru_cache(maxsize=None)
def _edge_kernel():
  return pl.kernel(
      _edge_body,
      out_type=jax.ShapeDtypeStruct(
          (_NC, _NS, _ROWS_PER_TILE, _D), jnp.float32),
      mesh=plsc.VectorSubcoreMesh(
          core_axis_name="c", subcore_axis_name="s",
          num_cores=_NC, num_subcores=_NS),
      scratch_types=(
          [pltpu.VMEM((_ERMAX, _CH), jnp.int32)] * 2
          + [pltpu.VMEM((_CH, _D), jnp.float32)] * _NBUF
          + [pltpu.VMEM_SHARED((_N, _D), jnp.float32)]
          + [pltpu.SemaphoreType.DMA] * (2 * _NBUF)
      ),
  )


def _dis_from_parts(degp):
  deg = degp[0, :, 0] + degp[1, :, 0]
  return jnp.where(deg > 0, lax.rsqrt(deg), 0.0)


def _support_body(x_ref, w_ref, degp_ref, o_ref):
  sup = jnp.dot(x_ref[...], w_ref[...], preferred_element_type=jnp.float32)
  dis = _dis_from_parts(degp_ref[...])
  o_ref[...] = sup * dis[:, None]


def _support_kernel(x, w, degp):
  return pl.pallas_call(
      _support_body,
      out_shape=jax.ShapeDtypeStruct((_N, _D), jnp.float32),
  )(x, w, degp)


def _finish_body(accp_ref, degp_ref, b_ref, o_ref):
  acc = accp_ref[0] + accp_ref[1]
  dis = _dis_from_parts(degp_ref[...])
  o_ref[...] = jnp.maximum(acc * dis[:, None] + b_ref[...], 0.0)


def _finish_kernel(accp, degp, b2d):
  return pl.pallas_call(
      _finish_body,
      out_shape=jax.ShapeDtypeStruct((_N, _D), jnp.float32),
  )(accp, degp, b2d)


@jax.jit
def kernel(x, edge_index, W, b):
  src = edge_index[0].reshape(_NW, _NCHUNK, _CH)
  dst = edge_index[1].reshape(_NW, _NCHUNK, _CH)
  zeros128 = jnp.zeros((_ROWS_PER_TILE, _D), jnp.float32)
  degp = _deg_kernel()(dst).reshape(_NC, _N, _DEGW)
  sup2 = _support_kernel(x, W, degp)
  accp = _edge_kernel()(sup2, src, dst, zeros128).reshape(_NC, _N, _D)
  return _finish_kernel(accp, degp, b.reshape(1, _D))


# speedup vs baseline: 26.6520x; 1.0143x over previous
import functools

import jax
import jax.numpy as jnp
from jax import lax
from jax.experimental import pallas as pl
from jax.experimental.pallas import tpu as pltpu
from jax.experimental.pallas import tpu_sc as plsc

_N = 10000
_E = 320000
_D = 128

_NC = 2
_NS = 16
_NW = _NC * _NS
_EPW = _E // _NW
_CH = 40
_NCHUNK = _EPW // _CH
_ROUNDS = ((0, 128), (128, 122))
_RMAX = 128
_EROUNDS = ((0, 64), (64, 64), (128, 64), (192, 58))
_ERMAX = 64
_NBUF = 6
_ROWS_PER_TILE = _N // _NS
_DEGW = 128


def _deg_body(dst_hbm, out_hbm, idx_v, ones_v, zbuf_v, deg_sh,
              sem_a, sem_b, sem_c, sem_d):
  c = lax.axis_index("c")
  s = lax.axis_index("s")
  wid = c * _NS + s

  def fill(i, _):
    ones_v[i, :] = jnp.ones((_DEGW,), jnp.float32)
    return 0

  lax.fori_loop(0, _CH, fill, 0, unroll=4)

  def zfill(i, _):
    zbuf_v[i, :] = jnp.zeros((_DEGW,), jnp.float32)
    return 0

  lax.fori_loop(0, 25, zfill, 0, unroll=4)

  def zcopy(q, _):
    pltpu.sync_copy(
        zbuf_v, deg_sh.at[pl.ds(s * _ROWS_PER_TILE + q * 25, 25)])
    return 0

  lax.fori_loop(0, _ROWS_PER_TILE // 25, zcopy, 0)
  plsc.subcore_barrier()

  sems = (sem_a, sem_b, sem_c, sem_d)

  def start(j, b):
    pltpu.async_copy(ones_v, deg_sh.at[idx_v.at[j]], sems[b], add=True)

  def wait(j, b):
    pltpu.make_async_copy(ones_v, deg_sh.at[idx_v.at[j]], sems[b]).wait()

  for off, nch in _EROUNDS:
    pltpu.sync_copy(dst_hbm.at[wid, pl.ds(off, nch)],
                    idx_v.at[pl.ds(0, nch)])
    nquad = nch // 4
    tail = nch % 4
    for b in range(4):
      start(b, b)

    def body(k, _):
      j = 4 * k
      for b in range(4):
        wait(j + b, b)

        @pl.when(j + 4 + b < nch)
        def _():
          start(j + 4 + b, b)

      return 0

    lax.fori_loop(0, nquad, body, 0)
    for b in range(tail):
      wait(nquad * 4 + b, b)

  plsc.subcore_barrier()
  pltpu.sync_copy(
      deg_sh.at[pl.ds(s * _ROWS_PER_TILE, _ROWS_PER_TILE)],
      out_hbm.at[c, s],
  )


@functools.lru_cache(maxsize=None)
def _deg_kernel():
  return pl.kernel(
      _deg_body,
      out_type=jax.ShapeDtypeStruct(
          (_NC, _NS, _ROWS_PER_TILE, _DEGW), jnp.float32),
      mesh=plsc.VectorSubcoreMesh(
          core_axis_name="c", subcore_axis_name="s",
          num_cores=_NC, num_subcores=_NS),
      scratch_types=[
          pltpu.VMEM((_ERMAX, _CH), jnp.int32),
          pltpu.VMEM((_CH, _DEGW), jnp.float32),
          pltpu.VMEM((25, _DEGW), jnp.float32),
          pltpu.VMEM_SHARED((_N, _DEGW), jnp.float32),
          pltpu.SemaphoreType.DMA,
          pltpu.SemaphoreType.DMA,
          pltpu.SemaphoreType.DMA,
          pltpu.SemaphoreType.DMA,
      ],
  )


def _edge_body(sup_hbm, src_hbm, dst_hbm, out_hbm, sidx_v, didx_v,
               rows0, rows1, rows2, rows3, rows4, rows5, acc_sh,
               gs0, gs1, gs2, gs3, gs4, gs5, ss0, ss1, ss2, ss3, ss4, ss5):
  c = lax.axis_index("c")
  s = lax.axis_index("s")
  wid = c * _NS + s
  rows = (rows0, rows1, rows2, rows3, rows4, rows5)
  gsem = (gs0, gs1, gs2, gs3, gs4, gs5)
  ssem = (ss0, ss1, ss2, ss3, ss4, ss5)

  def zfill(i, _):
    for q in range(_D // 16):
      rows0[i, pl.ds(q * 16, 16)] = jnp.zeros((16,), jnp.float32)
    return 0

  lax.fori_loop(0, _CH, zfill, 0, unroll=4)
  base = s * _ROWS_PER_TILE

  def zcopy(q, _):
    pltpu.sync_copy(rows0, acc_sh.at[pl.ds(base + q * _CH, _CH)])
    return 0

  lax.fori_loop(0, _ROWS_PER_TILE // _CH, zcopy, 0)
  tail_rows = _ROWS_PER_TILE % _CH
  pltpu.sync_copy(
      rows0.at[pl.ds(0, tail_rows)],
      acc_sh.at[pl.ds(base + _ROWS_PER_TILE - tail_rows, tail_rows)])
  plsc.subcore_barrier()

  def gstart(j, b):
    pltpu.async_copy(sup_hbm.at[sidx_v.at[j]], rows[b], gsem[b])

  def gwait(j, b):
    pltpu.make_async_copy(sup_hbm.at[sidx_v.at[j]], rows[b], gsem[b]).wait()

  def sstart(j, b):
    pltpu.async_copy(rows[b], acc_sh.at[didx_v.at[j]], ssem[b], add=True)

  def swait(j, b):
    pltpu.make_async_copy(rows[b], acc_sh.at[didx_v.at[j]], ssem[b]).wait()

  for off, nch in _EROUNDS:
    pltpu.sync_copy(src_hbm.at[wid, pl.ds(off, nch)],
                    sidx_v.at[pl.ds(0, nch)])
    pltpu.sync_copy(dst_hbm.at[wid, pl.ds(off, nch)],
                    didx_v.at[pl.ds(0, nch)])
    nquad = nch // _NBUF
    tail = nch % _NBUF
    for b in range(_NBUF):
      gstart(b, b)

    def body(k, _):
      j = _NBUF * k
      for b in range(_NBUF):
        gwait(j + b, b)
        sstart(j + b, b)
      for b in range(_NBUF):
        swait(j + b, b)

        @pl.when(j + _NBUF + b < nch)
        def _():
          gstart(j + _NBUF + b, b)

      return 0

    lax.fori_loop(0, nquad, body, 0)
    for b in range(tail):
      j = nquad * _NBUF + b
      gwait(j, b)
      sstart(j, b)
      swait(j, b)

  plsc.subcore_barrier()
  pltpu.sync_copy(
      acc_sh.at[pl.ds(s * _ROWS_PER_TILE, _ROWS_PER_TILE)],
      out_hbm.at[c, s],
  )


@functools.lru_cache(maxsize=None)
def _edge_kernel():
  return pl.kernel(
      _edge_body,
      out_type=jax.ShapeDtypeStruct(
          (_NC, _NS, _ROWS_PER_TILE, _D), jnp.float32),
      mesh=plsc.VectorSubcoreMesh(
          core_axis_name="c", subcore_axis_name="s",
          num_cores=_NC, num_subcores=_NS),
      scratch_types=(
          [pltpu.VMEM((_ERMAX, _CH), jnp.int32)] * 2
          + [pltpu.VMEM((_CH, _D), jnp.float32)] * _NBUF
          + [pltpu.VMEM_SHARED((_N, _D), jnp.float32)]
          + [pltpu.SemaphoreType.DMA] * (2 * _NBUF)
      ),
  )


def _dis_from_parts(degp):
  deg = degp[0, :, 0] + degp[1, :, 0]
  return jnp.where(deg > 0, lax.rsqrt(deg), 0.0)


def _support_body(x_ref, w_ref, degp_ref, o_ref):
  sup = jnp.dot(x_ref[...], w_ref[...], preferred_element_type=jnp.float32)
  dis = _dis_from_parts(degp_ref[...])
  o_ref[...] = sup * dis[:, None]


def _support_kernel(x, w, degp):
  return pl.pallas_call(
      _support_body,
      out_shape=jax.ShapeDtypeStruct((_N, _D), jnp.float32),
  )(x, w, degp)


def _finish_body(accp_ref, degp_ref, b_ref, o_ref):
  acc = accp_ref[0] + accp_ref[1]
  dis = _dis_from_parts(degp_ref[...])
  o_ref[...] = jnp.maximum(acc * dis[:, None] + b_ref[...], 0.0)


def _finish_kernel(accp, degp, b2d):
  return pl.pallas_call(
      _finish_body,
      out_shape=jax.ShapeDtypeStruct((_N, _D), jnp.float32),
  )(accp, degp, b2d)


@jax.jit
def kernel(x, edge_index, W, b):
  src = edge_index[0].reshape(_NW, _NCHUNK, _CH)
  dst = edge_index[1].reshape(_NW, _NCHUNK, _CH)
  degp = _deg_kernel()(dst).reshape(_NC, _N, _DEGW)
  sup2 = _support_kernel(x, W, degp)
  accp = _edge_kernel()(sup2, src, dst).reshape(_NC, _N, _D)
  return _finish_kernel(accp, degp, b.reshape(1, _D))
